# baseline (speedup 1.0000x reference)
import functools

import jax
import jax.numpy as jnp
from jax import lax
from jax.experimental import pallas as pl
from jax.experimental.pallas import tpu as pltpu
from jax.experimental.pallas import tpu_sc as plsc


def _sc_gather_t(idx, emb_t):
    D, V = emb_t.shape
    B = idx.shape[0]
    info = plsc.get_sparse_core_info()
    nw = info.num_cores * info.num_subcores
    d_per_w = D // nw
    mesh = plsc.VectorSubcoreMesh(core_axis_name="c", subcore_axis_name="s")

    @functools.partial(
        pl.kernel,
        mesh=mesh,
        out_type=jax.ShapeDtypeStruct((D, B), jnp.float32),
        scratch_types=[
            pltpu.VMEM((B,), jnp.int32),
            pltpu.VMEM((V,), jnp.float32),
            pltpu.VMEM((B,), jnp.float32),
            pltpu.VMEM((B,), jnp.float32),
            pltpu.SemaphoreType.DMA,
            pltpu.SemaphoreType.DMA,
            pltpu.SemaphoreType.DMA,
        ],
        compiler_params=pltpu.CompilerParams(needs_layout_passes=False),
    )
    def gather_kernel(idx_hbm, emb_hbm, out_hbm, idx_v, row_v, ot0, ot1,
                      sem_i, sem_r, sem_w):
        wid = lax.axis_index("s") * info.num_cores + lax.axis_index("c")
        ots = (ot0, ot1)
        ci = pltpu.async_copy(idx_hbm, idx_v, sem_i)
        cr = pltpu.async_copy(emb_hbm.at[wid * d_per_w], row_v, sem_r)
        ci.wait()
        wbs = []
        for r in range(d_per_w):
            d = wid * d_per_w + r
            cr.wait()
            ot = ots[r % 2]
            for j in range(B // 16):
                sl = pl.ds(j * 16, 16)
                ot[sl] = plsc.load_gather(row_v, [idx_v[sl]])
            if r + 1 < d_per_w:
                cr = pltpu.async_copy(emb_hbm.at[d + 1], row_v, sem_r)
            wbs.append(pltpu.async_copy(ot, out_hbm.at[d], sem_w))
        for wb in wbs:
            wb.wait()

    return gather_kernel(idx, emb_t)


_TILE_V = 2048


def _tc_matmul_t(wt, et):
    D, V = wt.shape
    B = et.shape[1]

    def body(wt_ref, et_ref, o_ref):
        o_ref[...] = lax.dot_general(
            wt_ref[...], et_ref[...],
            (((0,), (0,)), ((), ())),
            preferred_element_type=jnp.float32,
        )

    return pl.pallas_call(
        body,
        grid=(pl.cdiv(V, _TILE_V),),
        in_specs=[
            pl.BlockSpec((D, _TILE_V), lambda i: (0, i)),
            pl.BlockSpec((D, B), lambda i: (0, 0)),
        ],
        out_specs=pl.BlockSpec((_TILE_V, B), lambda i: (i, 0)),
        out_shape=jax.ShapeDtypeStruct((V, B), jnp.float32),
        compiler_params=pltpu.CompilerParams(
            dimension_semantics=("parallel",),
        ),
    )(wt, et)


def kernel(x, embed_weight, linear_weight):
    et = _sc_gather_t(x.astype(jnp.int32), embed_weight.T)
    out_t = _tc_matmul_t(linear_weight.T, et)
    return out_t.T


# speedup vs baseline: 1.0148x; 1.0148x over previous
import functools

import jax
import jax.numpy as jnp
from jax import lax
from jax.experimental import pallas as pl
from jax.experimental.pallas import tpu as pltpu
from jax.experimental.pallas import tpu_sc as plsc


def _sc_gather_t(idx, emb_t):
    D, V = emb_t.shape
    B = idx.shape[0]
    info = plsc.get_sparse_core_info()
    nw = info.num_cores * info.num_subcores
    d_per_w = D // nw
    mesh = plsc.VectorSubcoreMesh(core_axis_name="c", subcore_axis_name="s")

    @functools.partial(
        pl.kernel,
        mesh=mesh,
        out_type=jax.ShapeDtypeStruct((D, B), jnp.float32),
        scratch_types=[
            pltpu.VMEM((B,), jnp.int32),
            pltpu.VMEM((V,), jnp.float32),
            pltpu.VMEM((B,), jnp.float32),
            pltpu.VMEM((B,), jnp.float32),
            pltpu.SemaphoreType.DMA,
            pltpu.SemaphoreType.DMA,
            pltpu.SemaphoreType.DMA,
        ],
        compiler_params=pltpu.CompilerParams(needs_layout_passes=False),
    )
    def gather_kernel(idx_hbm, emb_hbm, out_hbm, idx_v, row_v, ot0, ot1,
                      sem_i, sem_r, sem_w):
        wid = lax.axis_index("s") * info.num_cores + lax.axis_index("c")
        ots = (ot0, ot1)
        ci = pltpu.async_copy(idx_hbm, idx_v, sem_i)
        cr = pltpu.async_copy(emb_hbm.at[wid * d_per_w], row_v, sem_r)
        ci.wait()
        wbs = []
        for r in range(d_per_w):
            d = wid * d_per_w + r
            cr.wait()
            ot = ots[r % 2]
            for j in range(B // 16):
                sl = pl.ds(j * 16, 16)
                ot[sl] = plsc.load_gather(row_v, [idx_v[sl]])
            if r + 1 < d_per_w:
                cr = pltpu.async_copy(emb_hbm.at[d + 1], row_v, sem_r)
            wbs.append(pltpu.async_copy(ot, out_hbm.at[d], sem_w))
        for wb in wbs:
            wb.wait()

    return gather_kernel(idx, emb_t)


_TILE_V = 4096


def _tc_matmul_t(wt, et):
    D, V = wt.shape
    B = et.shape[1]

    def body(wt_ref, et_ref, o_ref):
        o_ref[...] = lax.dot_general(
            wt_ref[...], et_ref[...],
            (((0,), (0,)), ((), ())),
            preferred_element_type=jnp.float32,
        )

    return pl.pallas_call(
        body,
        grid=(pl.cdiv(V, _TILE_V),),
        in_specs=[
            pl.BlockSpec((D, _TILE_V), lambda i: (0, i)),
            pl.BlockSpec((D, B), lambda i: (0, 0)),
        ],
        out_specs=pl.BlockSpec((_TILE_V, B), lambda i: (i, 0)),
        out_shape=jax.ShapeDtypeStruct((V, B), jnp.float32),
        compiler_params=pltpu.CompilerParams(
            dimension_semantics=("parallel",),
        ),
    )(wt, et)


def kernel(x, embed_weight, linear_weight):
    et = _sc_gather_t(x.astype(jnp.int32), embed_weight.T)
    out_t = _tc_matmul_t(linear_weight.T, et)
    return out_t.T
